# Initial kernel scaffold; baseline (speedup 1.0000x reference)
import jax
import jax.numpy as jnp
from jax.experimental import pallas as pl


def kernel(x, activations, indices):
    raise NotImplementedError("write your pallas kernel here")


# speedup vs baseline: 1.8234x; 1.8234x over previous
import jax
import jax.numpy as jnp
from jax.experimental import pallas as pl
from jax.experimental.pallas import tpu as pltpu

_ABLATION_VALUE = 10000000.0

_N, _C, _H, _W = 64, 768, 24, 24
_HW = _H * _W
_ROWS2D = _N * _C * _HW // 1024
_BLK_ROWS = 1024
_GRID1 = _ROWS2D // _BLK_ROWS


def _copy_min_body(x_ref, o_ref, vals_ref, acc_ref):
    i = pl.program_id(0)
    blk = x_ref[...]
    o_ref[...] = blk
    bmin = jnp.min(blk)

    @pl.when(i == 0)
    def _():
        acc_ref[0] = bmin

    @pl.when(i > 0)
    def _():
        acc_ref[0] = jnp.minimum(acc_ref[0], bmin)

    @pl.when(i == _GRID1 - 1)
    def _():
        def body(j, m):
            v = jnp.where(m == 0.0, jnp.float32(0.0), m - _ABLATION_VALUE)
            vals_ref[0, j] = v
            return v

        jax.lax.fori_loop(0, _N, body, acc_ref[0])


def _scatter_body(idx_ref, vals_ref, data_ref, o_ref):
    i = pl.program_id(0)
    del data_ref
    o_ref[...] = jnp.full((1, 1, _HW), vals_ref[0, i], jnp.float32)


def kernel(x, activations, indices):
    del x
    a2 = activations.reshape(_ROWS2D, 1024)
    copied, vals = pl.pallas_call(
        _copy_min_body,
        grid=(_GRID1,),
        in_specs=[pl.BlockSpec((_BLK_ROWS, 1024), lambda i: (i, 0))],
        out_specs=[
            pl.BlockSpec((_BLK_ROWS, 1024), lambda i: (i, 0)),
            pl.BlockSpec(memory_space=pltpu.SMEM),
        ],
        out_shape=[
            jax.ShapeDtypeStruct((_ROWS2D, 1024), jnp.float32),
            jax.ShapeDtypeStruct((1, _N), jnp.float32),
        ],
        scratch_shapes=[pltpu.SMEM((1,), jnp.float32)],
    )(a2)

    data = copied.reshape(_N * _C, 1, _HW)
    out = pl.pallas_call(
        _scatter_body,
        grid_spec=pltpu.PrefetchScalarGridSpec(
            num_scalar_prefetch=1,
            grid=(_N,),
            in_specs=[
                pl.BlockSpec(memory_space=pltpu.SMEM),
                pl.BlockSpec(
                    (1, 1, _HW), lambda i, idx_ref: (i * _C + idx_ref[i], 0, 0)
                ),
            ],
            out_specs=pl.BlockSpec(
                (1, 1, _HW), lambda i, idx_ref: (i * _C + idx_ref[i], 0, 0)
            ),
        ),
        out_shape=jax.ShapeDtypeStruct((_N * _C, 1, _HW), jnp.float32),
        input_output_aliases={2: 0},
    )(indices, vals, data)
    return out.reshape(_N, _C, _H, _W)


# speedup vs baseline: 24.5743x; 13.4769x over previous
import jax
import jax.numpy as jnp
from jax.experimental import pallas as pl
from jax.experimental.pallas import tpu as pltpu

_ABLATION_VALUE = 10000000.0

_N, _C, _H, _W = 64, 768, 24, 24
_HW = _H * _W
_ROWS = _N * _HW
_BLK_ROWS = 1024
_GRID1 = _ROWS // _BLK_ROWS


def _copy_min_body(x_ref, o_ref, vals_ref, acc_ref):
    i = pl.program_id(0)
    blk = x_ref[...]
    o_ref[...] = blk
    bmin = jnp.min(blk)

    @pl.when(i == 0)
    def _():
        acc_ref[0] = bmin

    @pl.when(i > 0)
    def _():
        acc_ref[0] = jnp.minimum(acc_ref[0], bmin)

    @pl.when(i == _GRID1 - 1)
    def _():
        def body(j, m):
            v = jnp.where(m == 0.0, jnp.float32(0.0), m - _ABLATION_VALUE)
            vals_ref[0, j] = v
            return v

        jax.lax.fori_loop(0, _N, body, acc_ref[0])


def _scatter_body(idx_ref, vals_ref, data_ref, o_ref):
    i = pl.program_id(0)
    col = idx_ref[i] % 128
    lanes = jax.lax.broadcasted_iota(jnp.int32, (_HW, 128), 1)
    o_ref[...] = jnp.where(lanes == col, vals_ref[0, i], data_ref[...])


def kernel(x, activations, indices):
    del x
    a2 = activations.transpose(0, 2, 3, 1).reshape(_ROWS, _C)
    copied, vals = pl.pallas_call(
        _copy_min_body,
        grid=(_GRID1,),
        in_specs=[pl.BlockSpec((_BLK_ROWS, _C), lambda i: (i, 0))],
        out_specs=[
            pl.BlockSpec((_BLK_ROWS, _C), lambda i: (i, 0)),
            pl.BlockSpec(memory_space=pltpu.SMEM),
        ],
        out_shape=[
            jax.ShapeDtypeStruct((_ROWS, _C), jnp.float32),
            jax.ShapeDtypeStruct((1, _N), jnp.float32),
        ],
        scratch_shapes=[pltpu.SMEM((1,), jnp.float32)],
    )(a2)

    out = pl.pallas_call(
        _scatter_body,
        grid_spec=pltpu.PrefetchScalarGridSpec(
            num_scalar_prefetch=1,
            grid=(_N,),
            in_specs=[
                pl.BlockSpec(memory_space=pltpu.SMEM),
                pl.BlockSpec((_HW, 128), lambda i, idx_ref: (i, idx_ref[i] // 128)),
            ],
            out_specs=pl.BlockSpec(
                (_HW, 128), lambda i, idx_ref: (i, idx_ref[i] // 128)
            ),
        ),
        out_shape=jax.ShapeDtypeStruct((_ROWS, _C), jnp.float32),
        input_output_aliases={2: 0},
    )(indices, vals, copied)
    return out.reshape(_N, _H, _W, _C).transpose(0, 3, 1, 2)
